# Initial kernel scaffold; baseline (speedup 1.0000x reference)
import jax
import jax.numpy as jnp
from jax.experimental import pallas as pl


def kernel(input_ids, embedding_table):
    raise NotImplementedError("write your pallas kernel here")


# speedup vs baseline: 3.0156x; 3.0156x over previous
import functools
import math

import jax
import jax.numpy as jnp
from jax import lax
from jax.experimental import pallas as pl
from jax.experimental.pallas import tpu as pltpu
from jax.experimental.pallas import tpu_sc as plsc

VOCAB_SIZE = 100000
D_MODEL = 64
SCALE = math.sqrt(D_MODEL)

NC = 2
NS = 16
NW = NC * NS
L = 16

G = 128
NB = 5
NROUNDS = None


def _embed_body(n_rounds, idx_hbm, table_hbm, out_hbm,
                idx_v, gbuf, obuf, gsem, osem):
  ng = n_rounds * NB
  wid = lax.axis_index("s") * NC + lax.axis_index("c")
  rw = ng * G
  row0 = wid * rw

  pltpu.sync_copy(idx_hbm.at[pl.ds(row0, rw)], idx_v)

  for b in range(NB):
    pltpu.async_copy(table_hbm.at[idx_v.at[pl.ds(b * G, G)]],
                     gbuf.at[b], gsem.at[b])

  def round_body(r):
    for b in range(NB):
      g = r * NB + b
      pltpu.make_async_copy(table_hbm.at[idx_v.at[pl.ds(0, G)]], gbuf.at[b],
                            gsem.at[b]).wait()
      @pl.when(r > 0)
      def _():
        pltpu.make_async_copy(obuf.at[b], out_hbm.at[pl.ds(0, G)],
                              osem.at[b]).wait()

      def mul_row(row):
        for c in range(D_MODEL // L):
          obuf[b, row, pl.ds(c * L, L)] = (
              gbuf[b, row, pl.ds(c * L, L)] * jnp.float32(SCALE))
      lax.fori_loop(0, G, lambda row, _: (mul_row(row), 0)[1], 0,
                    unroll=2)

      pltpu.async_copy(obuf.at[b], out_hbm.at[pl.ds(row0 + g * G, G)],
                       osem.at[b])

      @pl.when(r < n_rounds - 1)
      def _():
        pltpu.async_copy(table_hbm.at[idx_v.at[pl.ds((g + NB) * G, G)]],
                         gbuf.at[b], gsem.at[b])

  lax.fori_loop(0, n_rounds, lambda r, _: (round_body(r), 0)[1], 0)

  for b in range(NB):
    pltpu.make_async_copy(obuf.at[b], out_hbm.at[pl.ds(0, G)],
                          osem.at[b]).wait()


@functools.partial(jax.jit, static_argnums=())
def _sc_embed(flat_idx, table):
  n = flat_idx.shape[0]
  assert n % (NW * G) == 0
  ng = n // (NW * G)
  assert ng % NB == 0
  n_rounds = ng // NB

  mesh = plsc.VectorSubcoreMesh(core_axis_name="c", subcore_axis_name="s",
                                num_cores=NC, num_subcores=NS)
  run = pl.kernel(
      functools.partial(_embed_body, n_rounds),
      out_type=jax.ShapeDtypeStruct((n, D_MODEL), jnp.float32),
      mesh=mesh,
      scratch_types=[
          pltpu.VMEM((ng * G,), jnp.int32),
          pltpu.VMEM((NB, G, D_MODEL), jnp.float32),
          pltpu.VMEM((NB, G, D_MODEL), jnp.float32),
          pltpu.SemaphoreType.DMA((NB,)),
          pltpu.SemaphoreType.DMA((NB,)),
      ],
      compiler_params=pltpu.CompilerParams(use_tc_tiling_on_sc=False),
  )
  return run(flat_idx, table)


def kernel(input_ids, embedding_table):
  b, s = input_ids.shape
  flat_idx = input_ids.reshape(b * s).astype(jnp.int32)
  out = _sc_embed(flat_idx, embedding_table)
  return out.reshape(b, s, D_MODEL)


# speedup vs baseline: 3.0591x; 1.0144x over previous
import functools
import math

import jax
import jax.numpy as jnp
from jax import lax
from jax.experimental import pallas as pl
from jax.experimental.pallas import tpu as pltpu
from jax.experimental.pallas import tpu_sc as plsc

VOCAB_SIZE = 100000
D_MODEL = 64
SCALE = math.sqrt(D_MODEL)

NC = 2
NS = 16
NW = NC * NS

G = 128
NB = 10
LOOKAHEAD = 5


def _gather_body(n_rounds, ids_hbm, table_hbm, out_hbm, idx_v, gbuf,
                 gsem, osem):
  npw = n_rounds * NB
  wid = lax.axis_index("s") * NC + lax.axis_index("c")
  row0 = wid * npw * G

  pltpu.sync_copy(ids_hbm.at[pl.ds(row0, npw * G)], idx_v)

  def start_gather(g, b):
    pltpu.async_copy(table_hbm.at[idx_v.at[pl.ds(g * G, G)]], gbuf.at[b],
                     gsem.at[b])

  def wait_gather(b):
    pltpu.make_async_copy(table_hbm.at[idx_v.at[pl.ds(0, G)]], gbuf.at[b],
                          gsem.at[b]).wait()

  def start_out(g, b):
    pltpu.async_copy(gbuf.at[b],
                     out_hbm.at[pl.ds(row0 + g * G, G), pl.ds(0, D_MODEL)],
                     osem.at[b])

  def wait_out(b):
    pltpu.make_async_copy(gbuf.at[b],
                          out_hbm.at[pl.ds(0, G), pl.ds(0, D_MODEL)],
                          osem.at[b]).wait()

  for b in range(LOOKAHEAD):
    start_gather(b, b)

  def round_body(r):
    for j in range(NB):
      g = r * NB + j
      wait_gather(j)
      start_out(g, j)
      j2 = (j + LOOKAHEAD) % NB
      if j < NB - LOOKAHEAD:
        @pl.when(r > 0)
        def _():
          wait_out(j2)
        start_gather(g + LOOKAHEAD, j2)
      else:
        @pl.when(r < n_rounds - 1)
        def _():
          wait_out(j2)
          start_gather(g + LOOKAHEAD, j2)

  lax.fori_loop(0, n_rounds, lambda r, _: (round_body(r), 0)[1], 0)

  for b in range(NB):
    wait_out(b)


def _sc_gather(ids_flat, table):
  n = ids_flat.shape[0]
  assert n % (NW * G * NB) == 0
  n_rounds = n // (NW * G * NB)

  mesh = plsc.VectorSubcoreMesh(core_axis_name="c", subcore_axis_name="s",
                                num_cores=NC, num_subcores=NS)
  run = pl.kernel(
      functools.partial(_gather_body, n_rounds),
      out_type=jax.ShapeDtypeStruct((n, 2 * D_MODEL), jnp.float32),
      mesh=mesh,
      scratch_types=[
          pltpu.VMEM((n // NW,), jnp.int32),
          pltpu.VMEM((NB, G, D_MODEL), jnp.float32),
          pltpu.SemaphoreType.DMA((NB,)),
          pltpu.SemaphoreType.DMA((NB,)),
      ],
      compiler_params=pltpu.CompilerParams(use_tc_tiling_on_sc=False),
  )
  return run(ids_flat, table)


def _transform_kernel(in_hbm, out_ref, scratch, sem):
  step = pl.program_id(0)
  nsteps = pl.num_programs(0)
  rows_per_s = scratch.shape[1]
  slot = lax.rem(step, 2)

  def src(i):
    return in_hbm.at[pl.ds(i * rows_per_s, rows_per_s)]

  @pl.when(step == 0)
  def _():
    pltpu.make_async_copy(src(0), scratch.at[0], sem.at[0]).start()

  @pl.when(step + 1 < nsteps)
  def _():
    nxt = step + 1
    slot2 = lax.rem(nxt, 2)
    pltpu.make_async_copy(src(nxt), scratch.at[slot2], sem.at[slot2]).start()

  pltpu.make_async_copy(src(step), scratch.at[slot], sem.at[slot]).wait()

  scale = jnp.float32(SCALE)
  for t in range(8):
    xt = scratch[slot, pl.ds(t * G, G), pl.ds(0, D_MODEL)]
    y = xt.T * scale
    out_ref[0, :, t, :, :] = y.reshape(8, 8, G)


def _tc_transform(inter, s_len, b_len):
  return pl.pallas_call(
      _transform_kernel,
      out_shape=jax.ShapeDtypeStruct(
          (s_len, D_MODEL // 8, b_len // G, 8, G), jnp.float32),
      grid=(s_len,),
      in_specs=[pl.BlockSpec(memory_space=pl.ANY)],
      out_specs=pl.BlockSpec((1, D_MODEL // 8, b_len // G, 8, G),
                             lambda i: (i, 0, 0, 0, 0)),
      scratch_shapes=[
          pltpu.VMEM((2, b_len, 2 * D_MODEL), jnp.float32),
          pltpu.SemaphoreType.DMA((2,)),
      ],
  )(inter)


def kernel(input_ids, embedding_table):
  b, s = input_ids.shape
  ids_flat = input_ids.T.reshape(b * s).astype(jnp.int32)
  inter = _sc_gather(ids_flat, embedding_table)
  out5 = _tc_transform(inter, s, b)
  return out5.transpose(2, 4, 0, 1, 3).reshape(b, s, D_MODEL)


# speedup vs baseline: 3.8061x; 1.2442x over previous
import functools
import math

import jax
import jax.numpy as jnp
from jax import lax
from jax.experimental import pallas as pl
from jax.experimental.pallas import tpu as pltpu
from jax.experimental.pallas import tpu_sc as plsc

VOCAB_SIZE = 100000
D_MODEL = 64
SCALE = math.sqrt(D_MODEL)

NC = 2
NS = 16
NW = NC * NS

G = 128
NB = 10
LOOKAHEAD = 5


def _gather_body(n_rounds, ids_hbm, table_hbm, out_hbm, idx_v, gbuf,
                 gsem, osem):
  npw = n_rounds * NB
  wid = lax.axis_index("s") * NC + lax.axis_index("c")
  m0 = wid * npw

  pltpu.sync_copy(ids_hbm.at[pl.ds(m0 * G, npw * G)], idx_v)

  def out_row0(m):
    i = m // 64
    tb = lax.rem(m, 64) // 8
    j = lax.rem(m, 8)
    return i * 8192 + j * 1024 + tb * G

  def start_gather(g, b):
    pltpu.async_copy(table_hbm.at[idx_v.at[pl.ds(g * G, G)]], gbuf.at[b],
                     gsem.at[b])

  def wait_gather(b):
    pltpu.make_async_copy(table_hbm.at[idx_v.at[pl.ds(0, G)]], gbuf.at[b],
                          gsem.at[b]).wait()

  def start_out(g, b):
    pltpu.async_copy(gbuf.at[b],
                     out_hbm.at[pl.ds(out_row0(m0 + g), G),
                                pl.ds(0, D_MODEL)],
                     osem.at[b])

  def wait_out(b):
    pltpu.make_async_copy(gbuf.at[b],
                          out_hbm.at[pl.ds(0, G), pl.ds(0, D_MODEL)],
                          osem.at[b]).wait()

  for b in range(LOOKAHEAD):
    start_gather(b, b)

  def round_body(r):
    for j in range(NB):
      g = r * NB + j
      wait_gather(j)
      start_out(g, j)
      j2 = (j + LOOKAHEAD) % NB
      if j < NB - LOOKAHEAD:
        @pl.when(r > 0)
        def _():
          wait_out(j2)
        start_gather(g + LOOKAHEAD, j2)
      else:
        @pl.when(r < n_rounds - 1)
        def _():
          wait_out(j2)
          start_gather(g + LOOKAHEAD, j2)

  lax.fori_loop(0, n_rounds, lambda r, _: (round_body(r), 0)[1], 0)

  for b in range(NB):
    wait_out(b)


def _sc_gather(ids_flat, table):
  n = ids_flat.shape[0]
  assert n % (NW * G * NB) == 0
  n_rounds = n // (NW * G * NB)

  mesh = plsc.VectorSubcoreMesh(core_axis_name="c", subcore_axis_name="s",
                                num_cores=NC, num_subcores=NS)
  run = pl.kernel(
      functools.partial(_gather_body, n_rounds),
      out_type=jax.ShapeDtypeStruct((n, 2 * D_MODEL), jnp.float32),
      mesh=mesh,
      scratch_types=[
          pltpu.VMEM((n // NW,), jnp.int32),
          pltpu.VMEM((NB, G, D_MODEL), jnp.float32),
          pltpu.SemaphoreType.DMA((NB,)),
          pltpu.SemaphoreType.DMA((NB,)),
      ],
      compiler_params=pltpu.CompilerParams(use_tc_tiling_on_sc=False),
  )
  return run(ids_flat, table)


def _transform_kernel(in_hbm, out_ref, scratch, sem):
  step = pl.program_id(0)
  nsteps = pl.num_programs(0)
  rows_per_s = scratch.shape[1]
  nbuf = scratch.shape[0]
  slot = lax.rem(step, nbuf)

  def copy(i, sl):
    return pltpu.make_async_copy(
        in_hbm.at[pl.ds(i * rows_per_s, rows_per_s)],
        scratch.at[sl], sem.at[sl])

  @pl.when(step == 0)
  def _():
    for i in range(nbuf - 1):
      copy(i, i).start()

  @pl.when(step + nbuf - 1 < nsteps)
  def _():
    nxt = step + nbuf - 1
    slot2 = lax.rem(nxt, nbuf)
    copy(nxt, slot2).start()

  copy(step, slot).wait()

  scale = jnp.float32(SCALE)
  for t in range(8):
    xt = scratch[slot, pl.ds(t * G, G), pl.ds(0, D_MODEL)]
    y = xt.T * scale
    out_ref[0, :, t, :, :] = y.reshape(8, 8, G)


def _tc_transform(inter, s_len, b_len):
  return pl.pallas_call(
      _transform_kernel,
      out_shape=jax.ShapeDtypeStruct(
          (s_len, D_MODEL // 8, b_len // G, 8, G), jnp.float32),
      grid=(s_len,),
      in_specs=[pl.BlockSpec(memory_space=pl.ANY)],
      out_specs=pl.BlockSpec((1, D_MODEL // 8, b_len // G, 8, G),
                             lambda i: (i, 0, 0, 0, 0)),
      scratch_shapes=[
          pltpu.VMEM((4, b_len, 2 * D_MODEL), jnp.float32),
          pltpu.SemaphoreType.DMA((4,)),
      ],
  )(inter)


def kernel(input_ids, embedding_table):
  b, s = input_ids.shape
  ids_flat = (input_ids.T.astype(jnp.int32)
              .reshape(s // 8, 8, b // G, G)
              .transpose(0, 2, 1, 3)
              .reshape(b * s))
  inter = _sc_gather(ids_flat, embedding_table)
  out5 = _tc_transform(inter, s, b)
  return out5.transpose(2, 4, 0, 1, 3).reshape(b, s, D_MODEL)


# speedup vs baseline: 4.5215x; 1.1880x over previous
import functools
import math

import jax
import jax.numpy as jnp
from jax import lax
from jax.experimental import pallas as pl
from jax.experimental.pallas import tpu as pltpu
from jax.experimental.pallas import tpu_sc as plsc

VOCAB_SIZE = 100000
D_MODEL = 64
SCALE = math.sqrt(D_MODEL)

NC = 2
NS = 16
NW = NC * NS

G = 128
NB = 10
LOOKAHEAD = 5


def _gather_body(n_rounds, ids_hbm, table_hbm, out_hbm, idx_v, gbuf,
                 gsem, osem):
  npw = n_rounds * NB
  wid = lax.axis_index("s") * NC + lax.axis_index("c")
  m0 = wid * npw

  pltpu.sync_copy(ids_hbm.at[pl.ds(m0 * G, npw * G)], idx_v)

  def out_row0(m):
    i = m // 64
    tb = lax.rem(m, 64) // 8
    j = lax.rem(m, 8)
    return i * 8192 + j * 1024 + tb * G

  def start_gather(g, b):
    pltpu.async_copy(table_hbm.at[idx_v.at[pl.ds(g * G, G)]], gbuf.at[b],
                     gsem.at[b])

  def wait_gather(b):
    pltpu.make_async_copy(table_hbm.at[idx_v.at[pl.ds(0, G)]], gbuf.at[b],
                          gsem.at[b]).wait()

  def start_out(g, b):
    pltpu.async_copy(gbuf.at[b],
                     out_hbm.at[pl.ds(out_row0(m0 + g), G),
                                pl.ds(0, D_MODEL)],
                     osem.at[b])

  def wait_out(b):
    pltpu.make_async_copy(gbuf.at[b],
                          out_hbm.at[pl.ds(0, G), pl.ds(0, D_MODEL)],
                          osem.at[b]).wait()

  for b in range(LOOKAHEAD):
    start_gather(b, b)

  def round_body(r):
    for j in range(NB):
      g = r * NB + j
      wait_gather(j)
      start_out(g, j)
      j2 = (j + LOOKAHEAD) % NB
      if j < NB - LOOKAHEAD:
        @pl.when(r > 0)
        def _():
          wait_out(j2)
        start_gather(g + LOOKAHEAD, j2)
      else:
        @pl.when(r < n_rounds - 1)
        def _():
          wait_out(j2)
          start_gather(g + LOOKAHEAD, j2)

  lax.fori_loop(0, n_rounds, lambda r, _: (round_body(r), 0)[1], 0)

  for b in range(NB):
    wait_out(b)


def _sc_gather(ids_flat, table):
  n = ids_flat.shape[0]
  assert n % (NW * G * NB) == 0
  n_rounds = n // (NW * G * NB)

  mesh = plsc.VectorSubcoreMesh(core_axis_name="c", subcore_axis_name="s",
                                num_cores=NC, num_subcores=NS)
  run = pl.kernel(
      functools.partial(_gather_body, n_rounds),
      out_type=jax.ShapeDtypeStruct((n, 2 * D_MODEL), jnp.float32),
      mesh=mesh,
      scratch_types=[
          pltpu.VMEM((n // NW,), jnp.int32),
          pltpu.VMEM((NB, G, D_MODEL), jnp.float32),
          pltpu.SemaphoreType.DMA((NB,)),
          pltpu.SemaphoreType.DMA((NB,)),
      ],
      compiler_params=pltpu.CompilerParams(use_tc_tiling_on_sc=False),
  )
  return run(ids_flat, table)


def _transform_kernel(in_hbm, out_ref, scratch, sem):
  step = pl.program_id(0)
  nsteps = pl.num_programs(0)
  rows_per_s = scratch.shape[1]
  nbuf = scratch.shape[0]
  slot = lax.rem(step, nbuf)

  def copy(i, sl):
    return pltpu.make_async_copy(
        in_hbm.at[pl.ds(i * rows_per_s, rows_per_s)],
        scratch.at[sl], sem.at[sl])

  @pl.when(step == 0)
  def _():
    for i in range(nbuf - 1):
      copy(i, i).start()

  @pl.when(step + nbuf - 1 < nsteps)
  def _():
    nxt = step + nbuf - 1
    slot2 = lax.rem(nxt, nbuf)
    copy(nxt, slot2).start()

  copy(step, slot).wait()

  scale = jnp.float32(SCALE)
  n_sb = out_ref.shape[0]
  for sl in range(n_sb):
    for t in range(8):
      xt = scratch[slot, pl.ds(sl * 1024 + t * G, G),
                   pl.ds(0, D_MODEL)]
      y = xt.T * scale
      out_ref[sl, :, t, :, :] = y.reshape(8, 8, G)


def _tc_transform(inter, s_len, b_len):
  n_sb = 2
  return pl.pallas_call(
      _transform_kernel,
      out_shape=jax.ShapeDtypeStruct(
          (s_len, D_MODEL // 8, b_len // G, 8, G), jnp.float32),
      grid=(s_len // n_sb,),
      in_specs=[pl.BlockSpec(memory_space=pl.ANY)],
      out_specs=pl.BlockSpec((n_sb, D_MODEL // 8, b_len // G, 8, G),
                             lambda i: (i, 0, 0, 0, 0)),
      scratch_shapes=[
          pltpu.VMEM((6, n_sb * b_len, 2 * D_MODEL), jnp.float32),
          pltpu.SemaphoreType.DMA((6,)),
      ],
  )(inter)


def kernel(input_ids, embedding_table):
  b, s = input_ids.shape
  ids_flat = (input_ids.T.astype(jnp.int32)
              .reshape(s // 8, 8, b // G, G)
              .transpose(0, 2, 1, 3)
              .reshape(b * s))
  inter = _sc_gather(ids_flat, embedding_table)
  out5 = _tc_transform(inter, s, b)
  return out5.transpose(2, 4, 0, 1, 3).reshape(b, s, D_MODEL)


# speedup vs baseline: 4.7929x; 1.0600x over previous
import functools
import math

import jax
import jax.numpy as jnp
from jax import lax
from jax.experimental import pallas as pl
from jax.experimental.pallas import tpu as pltpu
from jax.experimental.pallas import tpu_sc as plsc

VOCAB_SIZE = 100000
D_MODEL = 64
SCALE = math.sqrt(D_MODEL)

NC = 2
NS = 16
NW = NC * NS

G = 128
NB = 10
LOOKAHEAD = 5


def _gather_body(n_rounds, ids_hbm, table_hbm, out_hbm, idx_v, gbuf,
                 gsem, osem):
  npw = n_rounds * NB
  wid = lax.axis_index("s") * NC + lax.axis_index("c")
  m0 = wid * npw

  pltpu.sync_copy(ids_hbm.at[pl.ds(m0 * G, npw * G)], idx_v)

  def out_row0(m):
    i = m // 64
    tb = lax.rem(m, 64) // 8
    j = lax.rem(m, 8)
    return i * 8192 + j * 1024 + tb * G

  def start_gather(g, b):
    pltpu.async_copy(table_hbm.at[idx_v.at[pl.ds(g * G, G)]], gbuf.at[b],
                     gsem.at[b])

  def wait_gather(b):
    pltpu.make_async_copy(table_hbm.at[idx_v.at[pl.ds(0, G)]], gbuf.at[b],
                          gsem.at[b]).wait()

  def start_out(g, b):
    pltpu.async_copy(gbuf.at[b],
                     out_hbm.at[pl.ds(out_row0(m0 + g), G),
                                pl.ds(0, D_MODEL)],
                     osem.at[b])

  def wait_out(b):
    pltpu.make_async_copy(gbuf.at[b],
                          out_hbm.at[pl.ds(0, G), pl.ds(0, D_MODEL)],
                          osem.at[b]).wait()

  for b in range(LOOKAHEAD):
    start_gather(b, b)

  def round_body(r):
    for j in range(NB):
      g = r * NB + j
      wait_gather(j)
      start_out(g, j)
      j2 = (j + LOOKAHEAD) % NB
      if j < NB - LOOKAHEAD:
        @pl.when(r > 0)
        def _():
          wait_out(j2)
        start_gather(g + LOOKAHEAD, j2)
      else:
        @pl.when(r < n_rounds - 1)
        def _():
          wait_out(j2)
          start_gather(g + LOOKAHEAD, j2)

  lax.fori_loop(0, n_rounds, lambda r, _: (round_body(r), 0)[1], 0)

  for b in range(NB):
    wait_out(b)


def _sc_gather(ids_flat, table):
  n = ids_flat.shape[0]
  assert n % (NW * G * NB) == 0
  n_rounds = n // (NW * G * NB)

  mesh = plsc.VectorSubcoreMesh(core_axis_name="c", subcore_axis_name="s",
                                num_cores=NC, num_subcores=NS)
  run = pl.kernel(
      functools.partial(_gather_body, n_rounds),
      out_type=jax.ShapeDtypeStruct((n, 2 * D_MODEL), jnp.float32),
      mesh=mesh,
      scratch_types=[
          pltpu.VMEM((n // NW,), jnp.int32),
          pltpu.VMEM((NB, G, D_MODEL), jnp.float32),
          pltpu.SemaphoreType.DMA((NB,)),
          pltpu.SemaphoreType.DMA((NB,)),
      ],
      compiler_params=pltpu.CompilerParams(use_tc_tiling_on_sc=False),
  )
  return run(ids_flat, table)


def _transform_kernel(in_hbm, out_ref, scratch, sem):
  step = pl.program_id(0)
  nsteps = pl.num_programs(0)
  rows_per_s = scratch.shape[1]
  nbuf = scratch.shape[0]
  slot = lax.rem(step, nbuf)

  def copy(i, sl):
    return pltpu.make_async_copy(
        in_hbm.at[pl.ds(i * rows_per_s, rows_per_s)],
        scratch.at[sl], sem.at[sl])

  @pl.when(step == 0)
  def _():
    for i in range(nbuf - 1):
      copy(i, i).start()

  @pl.when(step + nbuf - 1 < nsteps)
  def _():
    nxt = step + nbuf - 1
    slot2 = lax.rem(nxt, nbuf)
    copy(nxt, slot2).start()

  copy(step, slot).wait()

  scale = jnp.float32(SCALE)
  n_sb = out_ref.shape[0]
  for sl in range(n_sb):
    for t in range(8):
      xt = scratch[slot, pl.ds(sl * 1024 + t * G, G),
                   pl.ds(0, D_MODEL)]
      y = xt.T * scale
      out_ref[sl, :, t, :, :] = y.reshape(8, 8, G)


def _tc_transform(inter, s_len, b_len):
  n_sb = 4
  return pl.pallas_call(
      _transform_kernel,
      out_shape=jax.ShapeDtypeStruct(
          (s_len, D_MODEL // 8, b_len // G, 8, G), jnp.float32),
      grid=(s_len // n_sb,),
      in_specs=[pl.BlockSpec(memory_space=pl.ANY)],
      out_specs=pl.BlockSpec((n_sb, D_MODEL // 8, b_len // G, 8, G),
                             lambda i: (i, 0, 0, 0, 0)),
      scratch_shapes=[
          pltpu.VMEM((6, n_sb * b_len, 2 * D_MODEL), jnp.float32),
          pltpu.SemaphoreType.DMA((6,)),
      ],
  )(inter)


def kernel(input_ids, embedding_table):
  b, s = input_ids.shape
  ids_flat = (input_ids.T.astype(jnp.int32)
              .reshape(s // 8, 8, b // G, G)
              .transpose(0, 2, 1, 3)
              .reshape(b * s))
  inter = _sc_gather(ids_flat, embedding_table)
  out5 = _tc_transform(inter, s, b)
  return out5.transpose(2, 4, 0, 1, 3).reshape(b, s, D_MODEL)
